# Initial kernel scaffold; baseline (speedup 1.0000x reference)
import jax
import jax.numpy as jnp
from jax.experimental import pallas as pl


def kernel(x, edge_index, edge_attr, batch, global_feats, W_self_0, W_msg_0, W_edge_0, b_0, W_self_1, W_msg_1, W_edge_1, b_1, W_self_2, W_msg_2, W_edge_2, b_2, W_pol, b_pol, W_val, b_val):
    raise NotImplementedError("write your pallas kernel here")


# speedup vs baseline: 3.0038x; 3.0038x over previous
import functools

import jax
import jax.numpy as jnp
from jax import lax
from jax.experimental import pallas as pl
from jax.experimental.pallas import tpu as pltpu
from jax.experimental.pallas import tpu_sc as plsc

_NC = 2
_NS = 16
_K = 80
_BN = 400
_BE = 2000


def _edge_terms(edge_attr, w_cat, n_out):
  E, DE = edge_attr.shape
  Ht = w_cat.shape[1]
  H = Ht // n_out

  def body(ea_ref, w_ref, *out_refs):
    r = jnp.dot(ea_ref[...], w_ref[...], preferred_element_type=jnp.float32)
    for t, o in enumerate(out_refs):
      o[...] = r[:, t * H:(t + 1) * H]

  return pl.pallas_call(
      body,
      grid=(E // _BE,),
      in_specs=[pl.BlockSpec((_BE, DE), lambda i: (i, 0)),
                pl.BlockSpec((DE, Ht), lambda i: (0, 0))],
      out_specs=[pl.BlockSpec((_BE, H), lambda i: (i, 0))] * n_out,
      out_shape=[jax.ShapeDtypeStruct((E, H), jnp.float32)] * n_out,
  )(edge_attr, w_cat)


def _proj_in(x, wm, ws, b):
  N, D = x.shape
  H = wm.shape[1]

  def body(x_ref, wm_ref, ws_ref, b_ref, hm_ref, hsb_ref):
    xb = x_ref[...]
    hm_ref[...] = jnp.dot(xb, wm_ref[...], preferred_element_type=jnp.float32)
    hsb_ref[...] = (jnp.dot(xb, ws_ref[...], preferred_element_type=jnp.float32)
                    + b_ref[...])

  return pl.pallas_call(
      body,
      grid=(N // _BN,),
      in_specs=[pl.BlockSpec((_BN, D), lambda i: (i, 0)),
                pl.BlockSpec((D, H), lambda i: (0, 0)),
                pl.BlockSpec((D, H), lambda i: (0, 0)),
                pl.BlockSpec((1, H), lambda i: (0, 0))],
      out_specs=[pl.BlockSpec((_BN, H), lambda i: (i, 0))] * 2,
      out_shape=[jax.ShapeDtypeStruct((N, H), jnp.float32)] * 2,
  )(x, wm, ws, b.reshape(1, H))


def _proj_mid(hsb, agg, wm, ws, b):
  N, H = hsb.shape

  def body(hsb_ref, agg_ref, wm_ref, ws_ref, b_ref, hm_ref, hsb_out_ref):
    h = jnp.maximum(hsb_ref[...] + agg_ref[0] + agg_ref[1], 0.0)
    hm_ref[...] = jnp.dot(h, wm_ref[...], preferred_element_type=jnp.float32)
    hsb_out_ref[...] = (jnp.dot(h, ws_ref[...], preferred_element_type=jnp.float32)
                        + b_ref[...])

  return pl.pallas_call(
      body,
      grid=(N // _BN,),
      in_specs=[pl.BlockSpec((_BN, H), lambda i: (i, 0)),
                pl.BlockSpec((2, _BN, H), lambda i: (0, i, 0)),
                pl.BlockSpec((H, H), lambda i: (0, 0)),
                pl.BlockSpec((H, H), lambda i: (0, 0)),
                pl.BlockSpec((1, H), lambda i: (0, 0))],
      out_specs=[pl.BlockSpec((_BN, H), lambda i: (i, 0))] * 2,
      out_shape=[jax.ShapeDtypeStruct((N, H), jnp.float32)] * 2,
  )(hsb, agg, wm, ws, b.reshape(1, H))


def _final_head(hsb, agg, batch3, gf, wp, bp, wv, bv):
  N, H = hsb.shape
  B, GD = gf.shape
  A = wp.shape[1]
  nblk = N // _BN

  def body(hsb_ref, agg_ref, b3_ref, gf_ref, wp_ref, bp_ref, wv_ref, bv_ref,
           logits_ref, value_ref, sums_ref, counts_ref):
    i = pl.program_id(0)

    @pl.when(i == 0)
    def _init():
      sums_ref[...] = jnp.zeros_like(sums_ref)
      counts_ref[...] = jnp.zeros_like(counts_ref)

    h = jnp.maximum(hsb_ref[...] + agg_ref[0] + agg_ref[1], 0.0)
    brow = b3_ref[0]
    seg = lax.broadcasted_iota(jnp.int32, (B, _BN), 0)
    onehot_t = (seg == jnp.broadcast_to(brow, (B, _BN))).astype(jnp.float32)
    sums_ref[...] += jnp.dot(onehot_t, h, preferred_element_type=jnp.float32)
    counts_ref[...] += jnp.broadcast_to(
        jnp.sum(onehot_t, axis=1, keepdims=True), (B, H))

    @pl.when(i == nblk - 1)
    def _heads():
      pooled = sums_ref[...] / jnp.maximum(counts_ref[...], 1.0)
      gfv = gf_ref[...]
      wpv = wp_ref[...]
      wvv = wv_ref[...]
      logits_ref[...] = (
          jnp.dot(pooled, wpv[:H], preferred_element_type=jnp.float32)
          + jnp.dot(gfv, wpv[H:], preferred_element_type=jnp.float32)
          + bp_ref[...])
      value_ref[...] = (
          jnp.dot(pooled, wvv[:H], preferred_element_type=jnp.float32)
          + jnp.dot(gfv, wvv[H:], preferred_element_type=jnp.float32)
          + bv_ref[...])

  return pl.pallas_call(
      body,
      grid=(nblk,),
      in_specs=[pl.BlockSpec((_BN, H), lambda i: (i, 0)),
                pl.BlockSpec((2, _BN, H), lambda i: (0, i, 0)),
                pl.BlockSpec((1, 1, _BN), lambda i: (i, 0, 0)),
                pl.BlockSpec((B, GD), lambda i: (0, 0)),
                pl.BlockSpec((H + GD, A), lambda i: (0, 0)),
                pl.BlockSpec((1, A), lambda i: (0, 0)),
                pl.BlockSpec((H + GD, 1), lambda i: (0, 0)),
                pl.BlockSpec((1, 1), lambda i: (0, 0))],
      out_specs=[pl.BlockSpec((B, A), lambda i: (0, 0)),
                 pl.BlockSpec((B, 1), lambda i: (0, 0))],
      out_shape=[jax.ShapeDtypeStruct((B, A), jnp.float32),
                 jax.ShapeDtypeStruct((B, 1), jnp.float32)],
      scratch_shapes=[pltpu.VMEM((B, H), jnp.float32),
                      pltpu.VMEM((B, H), jnp.float32)],
  )(hsb, agg, batch3, gf, wp, bp.reshape(1, A), wv, bv.reshape(1, 1))


def _sc_edge_pass(hm, ew, src, dst, zeros):
  N, H = hm.shape
  Np = zeros.shape[0]
  E = src.shape[0]
  nw = _NC * _NS
  epw = E // nw
  nchunk = epw // _K
  rpt = Np // _NS
  hc = H // 16

  mesh = plsc.VectorSubcoreMesh(core_axis_name="c", subcore_axis_name="s")

  @functools.partial(
      pl.kernel,
      out_type=jax.ShapeDtypeStruct((_NC, Np, H), jnp.float32),
      mesh=mesh,
      scratch_types=[
          pltpu.VMEM((_K,), jnp.int32),
          pltpu.VMEM((_K,), jnp.int32),
          pltpu.VMEM((_K, H), jnp.float32),
          pltpu.VMEM((_K, H), jnp.float32),
          pltpu.VMEM_SHARED((Np, H), jnp.float32),
          pltpu.SemaphoreType.DMA,
          pltpu.SemaphoreType.DMA,
      ])
  def k(hm_hbm, ew_hbm, src_hbm, dst_hbm, z_hbm, agg_hbm,
        src_v, dst_v, gm_v, ew_v, agg_sh, sem1, sem2):
    cid = lax.axis_index("c")
    sid = lax.axis_index("s")
    wid = sid * _NC + cid
    pltpu.sync_copy(z_hbm.at[pl.ds(sid * rpt, rpt)],
                    agg_sh.at[pl.ds(sid * rpt, rpt)])
    plsc.subcore_barrier()

    ebase = wid * epw

    def chunk(i, carry):
      base = ebase + i * _K
      pltpu.sync_copy(src_hbm.at[pl.ds(base, _K)], src_v)
      pltpu.sync_copy(dst_hbm.at[pl.ds(base, _K)], dst_v)
      cp_ew = pltpu.async_copy(ew_hbm.at[pl.ds(base, _K)], ew_v, sem1)
      cp_gm = pltpu.async_copy(hm_hbm.at[src_v], gm_v, sem2)
      cp_ew.wait()
      cp_gm.wait()

      def row(r, rc):
        for c in range(hc):
          sl = pl.ds(c * 16, 16)
          gm_v[r, sl] = jnp.maximum(gm_v[r, sl] + ew_v[r, sl], 0.0)
        return rc

      lax.fori_loop(0, _K, row, 0)
      pltpu.sync_copy(gm_v, agg_sh.at[dst_v], add=True)
      return carry

    lax.fori_loop(0, nchunk, chunk, 0)
    plsc.subcore_barrier()
    pltpu.sync_copy(agg_sh.at[pl.ds(sid * rpt, rpt)],
                    agg_hbm.at[cid, pl.ds(sid * rpt, rpt)])

  return k(hm, ew, src, dst, zeros)


def kernel(x, edge_index, edge_attr, batch, global_feats,
           W_self_0, W_msg_0, W_edge_0, b_0,
           W_self_1, W_msg_1, W_edge_1, b_1,
           W_self_2, W_msg_2, W_edge_2, b_2,
           W_pol, b_pol, W_val, b_val):
  N, D = x.shape
  E = edge_index.shape[1]
  H = W_self_0.shape[1]
  B = global_feats.shape[0]

  src = edge_index[0]
  dst = edge_index[1]
  npad = ((N + 8 * _NS - 1) // (8 * _NS)) * (8 * _NS)
  zeros = jnp.zeros((npad, H), jnp.float32)
  batch3 = batch.reshape(N // _BN, 1, _BN)

  w_edge_cat = jnp.concatenate([W_edge_0, W_edge_1, W_edge_2], axis=1)
  ew0, ew1, ew2 = _edge_terms(edge_attr, w_edge_cat, 3)

  hm, hsb = _proj_in(x, W_msg_0, W_self_0, b_0)
  agg = _sc_edge_pass(hm, ew0, src, dst, zeros)
  hm, hsb = _proj_mid(hsb, agg, W_msg_1, W_self_1, b_1)
  agg = _sc_edge_pass(hm, ew1, src, dst, zeros)
  hm, hsb = _proj_mid(hsb, agg, W_msg_2, W_self_2, b_2)
  agg = _sc_edge_pass(hm, ew2, src, dst, zeros)

  logits, value = _final_head(hsb, agg, batch3, global_feats,
                              W_pol, b_pol, W_val, b_val)
  return logits, value


# speedup vs baseline: 4.7676x; 1.5872x over previous
import functools

import jax
import jax.numpy as jnp
from jax import lax
from jax.experimental import pallas as pl
from jax.experimental.pallas import tpu as pltpu
from jax.experimental.pallas import tpu_sc as plsc

_NC = 2
_NS = 16
_K = 80
_BN = 400
_BE = 2000


def _edge_terms(edge_attr, w_cat, n_out):
  E, DE = edge_attr.shape
  Ht = w_cat.shape[1]
  H = Ht // n_out

  def body(ea_ref, w_ref, *out_refs):
    r = jnp.dot(ea_ref[...], w_ref[...], preferred_element_type=jnp.float32)
    for t, o in enumerate(out_refs):
      o[...] = r[:, t * H:(t + 1) * H]

  return pl.pallas_call(
      body,
      grid=(E // _BE,),
      in_specs=[pl.BlockSpec((_BE, DE), lambda i: (i, 0)),
                pl.BlockSpec((DE, Ht), lambda i: (0, 0))],
      out_specs=[pl.BlockSpec((_BE, H), lambda i: (i, 0))] * n_out,
      out_shape=[jax.ShapeDtypeStruct((E, H), jnp.float32)] * n_out,
  )(edge_attr, w_cat)


def _proj_in(x, wm, ws, b):
  N, D = x.shape
  H = wm.shape[1]

  def body(x_ref, wm_ref, ws_ref, b_ref, hm_ref, hsb_ref):
    xb = x_ref[...]
    hm_ref[...] = jnp.dot(xb, wm_ref[...], preferred_element_type=jnp.float32)
    hsb_ref[...] = (jnp.dot(xb, ws_ref[...], preferred_element_type=jnp.float32)
                    + b_ref[...])

  return pl.pallas_call(
      body,
      grid=(N // _BN,),
      in_specs=[pl.BlockSpec((_BN, D), lambda i: (i, 0)),
                pl.BlockSpec((D, H), lambda i: (0, 0)),
                pl.BlockSpec((D, H), lambda i: (0, 0)),
                pl.BlockSpec((1, H), lambda i: (0, 0))],
      out_specs=[pl.BlockSpec((_BN, H), lambda i: (i, 0))] * 2,
      out_shape=[jax.ShapeDtypeStruct((N, H), jnp.float32)] * 2,
  )(x, wm, ws, b.reshape(1, H))


def _proj_mid(hsb, agg, wm, ws, b):
  N, H = hsb.shape

  def body(hsb_ref, agg_ref, wm_ref, ws_ref, b_ref, hm_ref, hsb_out_ref):
    h = jnp.maximum(hsb_ref[...] + agg_ref[0] + agg_ref[1], 0.0)
    hm_ref[...] = jnp.dot(h, wm_ref[...], preferred_element_type=jnp.float32)
    hsb_out_ref[...] = (jnp.dot(h, ws_ref[...], preferred_element_type=jnp.float32)
                        + b_ref[...])

  return pl.pallas_call(
      body,
      grid=(N // _BN,),
      in_specs=[pl.BlockSpec((_BN, H), lambda i: (i, 0)),
                pl.BlockSpec((2, _BN, H), lambda i: (0, i, 0)),
                pl.BlockSpec((H, H), lambda i: (0, 0)),
                pl.BlockSpec((H, H), lambda i: (0, 0)),
                pl.BlockSpec((1, H), lambda i: (0, 0))],
      out_specs=[pl.BlockSpec((_BN, H), lambda i: (i, 0))] * 2,
      out_shape=[jax.ShapeDtypeStruct((N, H), jnp.float32)] * 2,
  )(hsb, agg, wm, ws, b.reshape(1, H))


def _final_head(hsb, agg, batch3, gf, wp, bp, wv, bv):
  N, H = hsb.shape
  B, GD = gf.shape
  A = wp.shape[1]
  nblk = N // _BN

  def body(hsb_ref, agg_ref, b3_ref, gf_ref, wp_ref, bp_ref, wv_ref, bv_ref,
           logits_ref, value_ref, sums_ref, counts_ref):
    i = pl.program_id(0)

    @pl.when(i == 0)
    def _init():
      sums_ref[...] = jnp.zeros_like(sums_ref)
      counts_ref[...] = jnp.zeros_like(counts_ref)

    h = jnp.maximum(hsb_ref[...] + agg_ref[0] + agg_ref[1], 0.0)
    brow = b3_ref[0]
    seg = lax.broadcasted_iota(jnp.int32, (B, _BN), 0)
    onehot_t = (seg == jnp.broadcast_to(brow, (B, _BN))).astype(jnp.float32)
    sums_ref[...] += jnp.dot(onehot_t, h, preferred_element_type=jnp.float32)
    counts_ref[...] += jnp.broadcast_to(
        jnp.sum(onehot_t, axis=1, keepdims=True), (B, H))

    @pl.when(i == nblk - 1)
    def _heads():
      pooled = sums_ref[...] / jnp.maximum(counts_ref[...], 1.0)
      gfv = gf_ref[...]
      wpv = wp_ref[...]
      wvv = wv_ref[...]
      logits_ref[...] = (
          jnp.dot(pooled, wpv[:H], preferred_element_type=jnp.float32)
          + jnp.dot(gfv, wpv[H:], preferred_element_type=jnp.float32)
          + bp_ref[...])
      value_ref[...] = (
          jnp.dot(pooled, wvv[:H], preferred_element_type=jnp.float32)
          + jnp.dot(gfv, wvv[H:], preferred_element_type=jnp.float32)
          + bv_ref[...])

  return pl.pallas_call(
      body,
      grid=(nblk,),
      in_specs=[pl.BlockSpec((_BN, H), lambda i: (i, 0)),
                pl.BlockSpec((2, _BN, H), lambda i: (0, i, 0)),
                pl.BlockSpec((1, 1, _BN), lambda i: (i, 0, 0)),
                pl.BlockSpec((B, GD), lambda i: (0, 0)),
                pl.BlockSpec((H + GD, A), lambda i: (0, 0)),
                pl.BlockSpec((1, A), lambda i: (0, 0)),
                pl.BlockSpec((H + GD, 1), lambda i: (0, 0)),
                pl.BlockSpec((1, 1), lambda i: (0, 0))],
      out_specs=[pl.BlockSpec((B, A), lambda i: (0, 0)),
                 pl.BlockSpec((B, 1), lambda i: (0, 0))],
      out_shape=[jax.ShapeDtypeStruct((B, A), jnp.float32),
                 jax.ShapeDtypeStruct((B, 1), jnp.float32)],
      scratch_shapes=[pltpu.VMEM((B, H), jnp.float32),
                      pltpu.VMEM((B, H), jnp.float32)],
  )(hsb, agg, batch3, gf, wp, bp.reshape(1, A), wv, bv.reshape(1, 1))


def _sc_edge_pass(hm, ew, src, dst, zeros):
  N, H = hm.shape
  Np = zeros.shape[0]
  E = src.shape[0]
  nw = _NC * _NS
  epw = E // nw
  nchunk = epw // _K
  rpt = Np // _NS
  hc = H // 16

  npair = (nchunk - 1) // 2

  mesh = plsc.VectorSubcoreMesh(core_axis_name="c", subcore_axis_name="s")

  @functools.partial(
      pl.kernel,
      out_type=jax.ShapeDtypeStruct((_NC, Np, H), jnp.float32),
      mesh=mesh,
      scratch_types=[
          pltpu.VMEM((_K,), jnp.int32),
          pltpu.VMEM((_K,), jnp.int32),
          pltpu.VMEM((_K,), jnp.int32),
          pltpu.VMEM((_K,), jnp.int32),
          pltpu.VMEM((_K, H), jnp.float32),
          pltpu.VMEM((_K, H), jnp.float32),
          pltpu.VMEM((_K, H), jnp.float32),
          pltpu.VMEM((_K, H), jnp.float32),
          pltpu.VMEM_SHARED((Np, H), jnp.float32),
          pltpu.SemaphoreType.DMA,
          pltpu.SemaphoreType.DMA,
          pltpu.SemaphoreType.DMA,
          pltpu.SemaphoreType.DMA,
          pltpu.SemaphoreType.DMA,
          pltpu.SemaphoreType.DMA,
      ])
  def k(hm_hbm, ew_hbm, src_hbm, dst_hbm, z_hbm, agg_hbm,
        src_a, src_b, dst_a, dst_b, gm_a, gm_b, ew_a, ew_b, agg_sh,
        ii_a, ii_b, ee_a, ee_b, gg_a, gg_b):
    cid = lax.axis_index("c")
    sid = lax.axis_index("s")
    wid = sid * _NC + cid
    ebase = wid * epw
    srcs = (src_a, src_b)
    dsts = (dst_a, dst_b)
    gms = (gm_a, gm_b)
    ews = (ew_a, ew_b)
    iis = (ii_a, ii_b)
    ees = (ee_a, ee_b)
    ggs = (gg_a, gg_b)
    last = nchunk - 1

    def issue_idx(ch, t):
      base = ebase + ch * _K
      pltpu.async_copy(src_hbm.at[pl.ds(base, _K)], srcs[t], iis[t])
      pltpu.async_copy(dst_hbm.at[pl.ds(base, _K)], dsts[t], iis[t])

    def wait_idx(ch, t):
      base = ebase + ch * _K
      pltpu.make_async_copy(src_hbm.at[pl.ds(base, _K)], srcs[t], iis[t]).wait()
      pltpu.make_async_copy(dst_hbm.at[pl.ds(base, _K)], dsts[t], iis[t]).wait()

    def issue_streams(ch, t):
      base = ebase + ch * _K
      pltpu.async_copy(ew_hbm.at[pl.ds(base, _K)], ews[t], ees[t])
      pltpu.async_copy(hm_hbm.at[srcs[t]], gms[t], ggs[t])

    def wait_streams(ch, t):
      base = ebase + ch * _K
      pltpu.make_async_copy(ew_hbm.at[pl.ds(base, _K)], ews[t], ees[t]).wait()
      pltpu.make_async_copy(hm_hbm.at[srcs[t]], gms[t], ggs[t]).wait()

    def compute_scatter(t):
      gm_v = gms[t]
      ew_v = ews[t]

      @plsc.parallel_loop(0, _K, 1, unroll=2)
      def _rows(r):
        for c in range(hc):
          sl = pl.ds(c * 16, 16)
          gm_v[r, sl] = jnp.maximum(gm_v[r, sl] + ew_v[r, sl], 0.0)

      pltpu.sync_copy(gm_v, agg_sh.at[dsts[t]], add=True)

    pltpu.sync_copy(z_hbm.at[pl.ds(sid * rpt, rpt)],
                    agg_sh.at[pl.ds(sid * rpt, rpt)])
    plsc.subcore_barrier()

    issue_idx(0, 0)
    issue_idx(1, 1)
    wait_idx(0, 0)
    issue_streams(0, 0)

    def pair(p, carry):
      for t in (0, 1):
        ch = 2 * p + t
        oth = 1 - t
        wait_idx(ch + 1, oth)
        issue_streams(ch + 1, oth)
        wait_streams(ch, t)
        compute_scatter(t)
        issue_idx(jnp.minimum(ch + 2, last), t)
      return carry

    lax.fori_loop(0, npair, pair, 0)
    wait_idx(last, 1)
    wait_streams(last, 0)
    compute_scatter(0)

    plsc.subcore_barrier()
    pltpu.sync_copy(agg_sh.at[pl.ds(sid * rpt, rpt)],
                    agg_hbm.at[cid, pl.ds(sid * rpt, rpt)])

  return k(hm, ew, src, dst, zeros)


def kernel(x, edge_index, edge_attr, batch, global_feats,
           W_self_0, W_msg_0, W_edge_0, b_0,
           W_self_1, W_msg_1, W_edge_1, b_1,
           W_self_2, W_msg_2, W_edge_2, b_2,
           W_pol, b_pol, W_val, b_val):
  N, D = x.shape
  E = edge_index.shape[1]
  H = W_self_0.shape[1]
  B = global_feats.shape[0]

  src = edge_index[0]
  dst = edge_index[1]
  npad = ((N + 8 * _NS - 1) // (8 * _NS)) * (8 * _NS)
  zeros = jnp.zeros((npad, H), jnp.float32)
  batch3 = batch.reshape(N // _BN, 1, _BN)

  ew0, = _edge_terms(edge_attr, W_edge_0, 1)
  hm, hsb = _proj_in(x, W_msg_0, W_self_0, b_0)
  agg = _sc_edge_pass(hm, ew0, src, dst, zeros)
  ew1, = _edge_terms(edge_attr, W_edge_1, 1)
  hm, hsb = _proj_mid(hsb, agg, W_msg_1, W_self_1, b_1)
  agg = _sc_edge_pass(hm, ew1, src, dst, zeros)
  ew2, = _edge_terms(edge_attr, W_edge_2, 1)
  hm, hsb = _proj_mid(hsb, agg, W_msg_2, W_self_2, b_2)
  agg = _sc_edge_pass(hm, ew2, src, dst, zeros)

  logits, value = _final_head(hsb, agg, batch3, global_feats,
                              W_pol, b_pol, W_val, b_val)
  return logits, value


# speedup vs baseline: 5.0374x; 1.0566x over previous
import functools

import jax
import jax.numpy as jnp
from jax import lax
from jax.experimental import pallas as pl
from jax.experimental.pallas import tpu as pltpu
from jax.experimental.pallas import tpu_sc as plsc

_NC = 2
_NS = 16
_K = 80
_BN = 400
_BE = 2000


def _edge_terms(edge_attr, w_cat, n_out):
  E, DE = edge_attr.shape
  Ht = w_cat.shape[1]
  H = Ht // n_out

  def body(ea_ref, w_ref, *out_refs):
    r = jnp.dot(ea_ref[...], w_ref[...], preferred_element_type=jnp.float32)
    for t, o in enumerate(out_refs):
      o[...] = r[:, t * H:(t + 1) * H]

  return pl.pallas_call(
      body,
      grid=(E // _BE,),
      in_specs=[pl.BlockSpec((_BE, DE), lambda i: (i, 0)),
                pl.BlockSpec((DE, Ht), lambda i: (0, 0))],
      out_specs=[pl.BlockSpec((_BE, H), lambda i: (i, 0))] * n_out,
      out_shape=[jax.ShapeDtypeStruct((E, H), jnp.float32)] * n_out,
  )(edge_attr, w_cat)


def _proj_in(x, wm, ws, b):
  N, D = x.shape
  H = wm.shape[1]

  def body(x_ref, wm_ref, ws_ref, b_ref, hm_ref, hsb_ref):
    xb = x_ref[...]
    hm_ref[...] = jnp.dot(xb, wm_ref[...], preferred_element_type=jnp.float32)
    hsb_ref[...] = (jnp.dot(xb, ws_ref[...], preferred_element_type=jnp.float32)
                    + b_ref[...])

  return pl.pallas_call(
      body,
      grid=(N // _BN,),
      in_specs=[pl.BlockSpec((_BN, D), lambda i: (i, 0)),
                pl.BlockSpec((D, H), lambda i: (0, 0)),
                pl.BlockSpec((D, H), lambda i: (0, 0)),
                pl.BlockSpec((1, H), lambda i: (0, 0))],
      out_specs=[pl.BlockSpec((_BN, H), lambda i: (i, 0))] * 2,
      out_shape=[jax.ShapeDtypeStruct((N, H), jnp.float32)] * 2,
  )(x, wm, ws, b.reshape(1, H))


def _proj_mid(hsb, agg, wm, ws, b):
  N, H = hsb.shape

  def body(hsb_ref, agg_ref, wm_ref, ws_ref, b_ref, hm_ref, hsb_out_ref):
    h = jnp.maximum(hsb_ref[...] + agg_ref[0] + agg_ref[1], 0.0)
    hm_ref[...] = jnp.dot(h, wm_ref[...], preferred_element_type=jnp.float32)
    hsb_out_ref[...] = (jnp.dot(h, ws_ref[...], preferred_element_type=jnp.float32)
                        + b_ref[...])

  return pl.pallas_call(
      body,
      grid=(N // _BN,),
      in_specs=[pl.BlockSpec((_BN, H), lambda i: (i, 0)),
                pl.BlockSpec((2, _BN, H), lambda i: (0, i, 0)),
                pl.BlockSpec((H, H), lambda i: (0, 0)),
                pl.BlockSpec((H, H), lambda i: (0, 0)),
                pl.BlockSpec((1, H), lambda i: (0, 0))],
      out_specs=[pl.BlockSpec((_BN, H), lambda i: (i, 0))] * 2,
      out_shape=[jax.ShapeDtypeStruct((N, H), jnp.float32)] * 2,
  )(hsb, agg, wm, ws, b.reshape(1, H))


def _final_head(hsb, agg, batch3, gf, wp, bp, wv, bv):
  N, H = hsb.shape
  B, GD = gf.shape
  A = wp.shape[1]
  nblk = N // _BN

  def body(hsb_ref, agg_ref, b3_ref, gf_ref, wp_ref, bp_ref, wv_ref, bv_ref,
           logits_ref, value_ref, sums_ref, counts_ref):
    i = pl.program_id(0)

    @pl.when(i == 0)
    def _init():
      sums_ref[...] = jnp.zeros_like(sums_ref)
      counts_ref[...] = jnp.zeros_like(counts_ref)

    h = jnp.maximum(hsb_ref[...] + agg_ref[0] + agg_ref[1], 0.0)
    brow = b3_ref[0]
    seg = lax.broadcasted_iota(jnp.int32, (B, _BN), 0)
    onehot_t = (seg == jnp.broadcast_to(brow, (B, _BN))).astype(jnp.float32)
    sums_ref[...] += jnp.dot(onehot_t, h, preferred_element_type=jnp.float32)
    counts_ref[...] += jnp.broadcast_to(
        jnp.sum(onehot_t, axis=1, keepdims=True), (B, H))

    @pl.when(i == nblk - 1)
    def _heads():
      pooled = sums_ref[...] / jnp.maximum(counts_ref[...], 1.0)
      gfv = gf_ref[...]
      wpv = wp_ref[...]
      wvv = wv_ref[...]
      logits_ref[...] = (
          jnp.dot(pooled, wpv[:H], preferred_element_type=jnp.float32)
          + jnp.dot(gfv, wpv[H:], preferred_element_type=jnp.float32)
          + bp_ref[...])
      value_ref[...] = (
          jnp.dot(pooled, wvv[:H], preferred_element_type=jnp.float32)
          + jnp.dot(gfv, wvv[H:], preferred_element_type=jnp.float32)
          + bv_ref[...])

  return pl.pallas_call(
      body,
      grid=(nblk,),
      in_specs=[pl.BlockSpec((_BN, H), lambda i: (i, 0)),
                pl.BlockSpec((2, _BN, H), lambda i: (0, i, 0)),
                pl.BlockSpec((1, 1, _BN), lambda i: (i, 0, 0)),
                pl.BlockSpec((B, GD), lambda i: (0, 0)),
                pl.BlockSpec((H + GD, A), lambda i: (0, 0)),
                pl.BlockSpec((1, A), lambda i: (0, 0)),
                pl.BlockSpec((H + GD, 1), lambda i: (0, 0)),
                pl.BlockSpec((1, 1), lambda i: (0, 0))],
      out_specs=[pl.BlockSpec((B, A), lambda i: (0, 0)),
                 pl.BlockSpec((B, 1), lambda i: (0, 0))],
      out_shape=[jax.ShapeDtypeStruct((B, A), jnp.float32),
                 jax.ShapeDtypeStruct((B, 1), jnp.float32)],
      scratch_shapes=[pltpu.VMEM((B, H), jnp.float32),
                      pltpu.VMEM((B, H), jnp.float32)],
  )(hsb, agg, batch3, gf, wp, bp.reshape(1, A), wv, bv.reshape(1, 1))


def _sc_edge_pass(hm, ew, src, dst, zeros):
  N, H = hm.shape
  Np = zeros.shape[0]
  E = src.shape[0]
  nw = _NC * _NS
  epw = E // nw
  nchunk = epw // _K
  rpt = Np // _NS
  hc = H // 16

  assert nchunk % 2 == 1 and nchunk >= 3

  mesh = plsc.VectorSubcoreMesh(core_axis_name="c", subcore_axis_name="s")

  @functools.partial(
      pl.kernel,
      out_type=jax.ShapeDtypeStruct((_NC, Np, H), jnp.float32),
      mesh=mesh,
      scratch_types=[
          pltpu.VMEM((_K,), jnp.int32),
          pltpu.VMEM((_K,), jnp.int32),
          pltpu.VMEM((_K,), jnp.int32),
          pltpu.VMEM((_K,), jnp.int32),
          pltpu.VMEM((_K,), jnp.int32),
          pltpu.VMEM((_K,), jnp.int32),
          pltpu.VMEM((_K, H), jnp.float32),
          pltpu.VMEM((_K, H), jnp.float32),
          pltpu.VMEM((_K, H), jnp.float32),
          pltpu.VMEM((_K, H), jnp.float32),
          pltpu.VMEM_SHARED((Np, H), jnp.float32),
          pltpu.SemaphoreType.DMA,
          pltpu.SemaphoreType.DMA,
          pltpu.SemaphoreType.DMA,
          pltpu.SemaphoreType.DMA,
          pltpu.SemaphoreType.DMA,
          pltpu.SemaphoreType.DMA,
          pltpu.SemaphoreType.DMA,
          pltpu.SemaphoreType.DMA,
      ])
  def k(hm_hbm, ew_hbm, src_hbm, dst_hbm, z_hbm, agg_hbm,
        src_a, src_b, dst_a, dst_b, ds_a, ds_b, gm_a, gm_b, ew_a, ew_b,
        agg_sh, ii_a, ii_b, ee_a, ee_b, gg_a, gg_b, ss_a, ss_b):
    cid = lax.axis_index("c")
    sid = lax.axis_index("s")
    wid = sid * _NC + cid
    ebase = wid * epw
    srcs = (src_a, src_b)
    dsts = (dst_a, dst_b)
    dscs = (ds_a, ds_b)
    gms = (gm_a, gm_b)
    ews = (ew_a, ew_b)
    iis = (ii_a, ii_b)
    ees = (ee_a, ee_b)
    ggs = (gg_a, gg_b)
    sss = (ss_a, ss_b)
    last = nchunk - 1

    def issue_idx(ch, t):
      base = ebase + ch * _K
      pltpu.async_copy(src_hbm.at[pl.ds(base, _K)], srcs[t], iis[t])
      pltpu.async_copy(dst_hbm.at[pl.ds(base, _K)], dsts[t], iis[t])

    def wait_idx(ch, t):
      base = ebase + ch * _K
      pltpu.make_async_copy(src_hbm.at[pl.ds(base, _K)], srcs[t], iis[t]).wait()
      pltpu.make_async_copy(dst_hbm.at[pl.ds(base, _K)], dsts[t], iis[t]).wait()

    def issue_streams(ch, t):
      base = ebase + ch * _K
      pltpu.async_copy(ew_hbm.at[pl.ds(base, _K)], ews[t], ees[t])
      pltpu.async_copy(hm_hbm.at[srcs[t]], gms[t], ggs[t])

    def wait_streams(ch, t):
      base = ebase + ch * _K
      pltpu.make_async_copy(ew_hbm.at[pl.ds(base, _K)], ews[t], ees[t]).wait()
      pltpu.make_async_copy(hm_hbm.at[srcs[t]], gms[t], ggs[t]).wait()

    def compute_scatter(t):
      gm_v = gms[t]
      ew_v = ews[t]
      dsc = dscs[t]
      for r2 in range(_K // 16):
        sl = pl.ds(r2 * 16, 16)
        dsc[sl] = dsts[t][sl]

      @plsc.parallel_loop(0, _K, 1, unroll=2)
      def _rows(r):
        for c in range(hc):
          sl = pl.ds(c * 16, 16)
          gm_v[r, sl] = jnp.maximum(gm_v[r, sl] + ew_v[r, sl], 0.0)

      pltpu.async_copy(gm_v, agg_sh.at[dsc], sss[t], add=True)

    def wait_scatter(t):
      pltpu.make_async_copy(gms[t], agg_sh.at[dscs[t]], sss[t]).wait()

    pltpu.sync_copy(z_hbm.at[pl.ds(sid * rpt, rpt)],
                    agg_sh.at[pl.ds(sid * rpt, rpt)])
    plsc.subcore_barrier()

    issue_idx(0, 0)
    issue_idx(1, 1)
    wait_idx(0, 0)
    issue_streams(0, 0)
    wait_idx(1, 1)
    issue_streams(1, 1)
    wait_streams(0, 0)
    compute_scatter(0)
    issue_idx(2, 0)

    def pair(p, carry):
      for j in (0, 1):
        ch = 2 * p + 1 + j
        t = 1 - j
        oth = 1 - t
        wait_idx(ch + 1, oth)
        wait_scatter(oth)
        issue_streams(ch + 1, oth)
        wait_streams(ch, t)
        compute_scatter(t)
        issue_idx(ch + 2, t)
      return carry

    lax.fori_loop(0, (nchunk - 3) // 2, pair, 0)
    tl = (last - 1) % 2
    wait_idx(last, 1 - tl)
    wait_scatter(1 - tl)
    issue_streams(last, 1 - tl)
    wait_streams(last - 1, tl)
    compute_scatter(tl)
    wait_streams(last, 1 - tl)
    compute_scatter(1 - tl)
    wait_scatter(tl)
    wait_scatter(1 - tl)

    plsc.subcore_barrier()
    pltpu.sync_copy(agg_sh.at[pl.ds(sid * rpt, rpt)],
                    agg_hbm.at[cid, pl.ds(sid * rpt, rpt)])

  return k(hm, ew, src, dst, zeros)


def kernel(x, edge_index, edge_attr, batch, global_feats,
           W_self_0, W_msg_0, W_edge_0, b_0,
           W_self_1, W_msg_1, W_edge_1, b_1,
           W_self_2, W_msg_2, W_edge_2, b_2,
           W_pol, b_pol, W_val, b_val):
  N, D = x.shape
  E = edge_index.shape[1]
  H = W_self_0.shape[1]
  B = global_feats.shape[0]

  src = edge_index[0]
  dst = edge_index[1]
  npad = ((N + 8 * _NS - 1) // (8 * _NS)) * (8 * _NS)
  zeros = jnp.zeros((npad, H), jnp.float32)
  batch3 = batch.reshape(N // _BN, 1, _BN)

  ew0, = _edge_terms(edge_attr, W_edge_0, 1)
  hm, hsb = _proj_in(x, W_msg_0, W_self_0, b_0)
  agg = _sc_edge_pass(hm, ew0, src, dst, zeros)
  ew1, = _edge_terms(edge_attr, W_edge_1, 1)
  hm, hsb = _proj_mid(hsb, agg, W_msg_1, W_self_1, b_1)
  agg = _sc_edge_pass(hm, ew1, src, dst, zeros)
  ew2, = _edge_terms(edge_attr, W_edge_2, 1)
  hm, hsb = _proj_mid(hsb, agg, W_msg_2, W_self_2, b_2)
  agg = _sc_edge_pass(hm, ew2, src, dst, zeros)

  logits, value = _final_head(hsb, agg, batch3, global_feats,
                              W_pol, b_pol, W_val, b_val)
  return logits, value
